# Initial kernel scaffold; baseline (speedup 1.0000x reference)
import functools

import jax
import jax.numpy as jnp
from jax import lax
from jax.experimental import pallas as pl
from jax.experimental.pallas import tpu as pltpu
from jax.experimental.pallas import tpu_sc as plsc

VOCAB = 1000000
DHATU_VOCAB = 10000
EMBED_DIM = 64
HALF = 32

NC = 2
NS = 16
NW = NC * NS

N_TOKENS = 16384 * 50
PER_W = N_TOKENS // NW
CHUNK = 512
SUB = 128
N_CHUNKS = PER_W // CHUNK


def _sc_gather(ids, word_table, dhatu_table):
    mesh = plsc.VectorSubcoreMesh(core_axis_name="c", subcore_axis_name="s")

    @functools.partial(
        pl.kernel,
        out_type=(
            jax.ShapeDtypeStruct((N_TOKENS, HALF), jnp.float32),
            jax.ShapeDtypeStruct((N_TOKENS, HALF), jnp.float32),
        ),
        mesh=mesh,
        scratch_types=[
            pltpu.VMEM((PER_W,), jnp.int32),
            pltpu.VMEM((PER_W,), jnp.int32),
            pltpu.VMEM((CHUNK, HALF), jnp.float32),
            pltpu.VMEM((CHUNK, HALF), jnp.float32),
            pltpu.SemaphoreType.DMA,
            pltpu.SemaphoreType.DMA,
        ],
    )
    def gather_kernel(ids_hbm, wt_hbm, dt_hbm, wout_hbm, dout_hbm,
                      idx_v, didx_v, wrows, drows, wsem, dsem):
        wid = lax.axis_index("s") * NC + lax.axis_index("c")
        base = wid * PER_W
        pltpu.sync_copy(ids_hbm.at[pl.ds(base, PER_W)], idx_v)

        def mod_body(i, carry):
            v = idx_v[pl.ds(i * 16, 16)]
            didx_v[pl.ds(i * 16, 16)] = lax.rem(v, DHATU_VOCAB)
            return carry

        lax.fori_loop(0, PER_W // 16, mod_body, 0)

        def chunk_body(ci, carry):
            off = ci * CHUNK
            handles = []
            for j in range(CHUNK // SUB):
                hw = pltpu.async_copy(
                    wt_hbm.at[idx_v.at[pl.ds(off + j * SUB, SUB)]],
                    wrows.at[pl.ds(j * SUB, SUB)], wsem)
                hd = pltpu.async_copy(
                    dt_hbm.at[didx_v.at[pl.ds(off + j * SUB, SUB)]],
                    drows.at[pl.ds(j * SUB, SUB)], dsem)
                handles.append((hw, hd))
            for hw, hd in handles:
                hw.wait()
                hd.wait()
            pltpu.sync_copy(wrows, wout_hbm.at[pl.ds(base + off, CHUNK)])
            pltpu.sync_copy(drows, dout_hbm.at[pl.ds(base + off, CHUNK)])
            return carry

        lax.fori_loop(0, N_CHUNKS, chunk_body, 0)

    return gather_kernel(ids, word_table, dhatu_table)


_SQRT_HALF = 0.7071067811865476


def _dense_body(w_ref, d_ref, w1t_ref, w2t_ref, b_ref, g_ref, bt_ref, o_ref):
    y = (jnp.dot(w_ref[...], w1t_ref[...], preferred_element_type=jnp.float32)
         + jnp.dot(d_ref[...], w2t_ref[...], preferred_element_type=jnp.float32)
         + b_ref[...])
    mu = jnp.mean(y, axis=-1, keepdims=True)
    var = jnp.mean((y - mu) ** 2, axis=-1, keepdims=True)
    yn = (y - mu) / jnp.sqrt(var + 1e-5) * g_ref[...] + bt_ref[...]
    o_ref[...] = yn * 0.5 * (1.0 + lax.erf(yn * _SQRT_HALF))


def _tc_dense(w_emb, d_emb, W, b, gamma, beta):
    BK = 2048
    grid = (N_TOKENS // BK,)
    w1t = W[:, :HALF].T
    w2t = W[:, HALF:].T
    return pl.pallas_call(
        _dense_body,
        grid=grid,
        in_specs=[
            pl.BlockSpec((BK, HALF), lambda i: (i, 0)),
            pl.BlockSpec((BK, HALF), lambda i: (i, 0)),
            pl.BlockSpec((HALF, EMBED_DIM), lambda i: (0, 0)),
            pl.BlockSpec((HALF, EMBED_DIM), lambda i: (0, 0)),
            pl.BlockSpec((1, EMBED_DIM), lambda i: (0, 0)),
            pl.BlockSpec((1, EMBED_DIM), lambda i: (0, 0)),
            pl.BlockSpec((1, EMBED_DIM), lambda i: (0, 0)),
        ],
        out_specs=pl.BlockSpec((BK, EMBED_DIM), lambda i: (i, 0)),
        out_shape=jax.ShapeDtypeStruct((N_TOKENS, EMBED_DIM), jnp.float32),
    )(w_emb, d_emb, w1t, w2t, b.reshape(1, EMBED_DIM),
      gamma.reshape(1, EMBED_DIM), beta.reshape(1, EMBED_DIM))


def kernel(word_ids, word_table, dhatu_table, W, b, gamma, beta):
    B, L = word_ids.shape
    ids = word_ids.reshape(-1)
    w_emb, d_emb = _sc_gather(ids, word_table, dhatu_table)
    y = _tc_dense(w_emb, d_emb, W, b, gamma, beta)
    return y.reshape(B, L, EMBED_DIM)


# speedup vs baseline: 10.1303x; 10.1303x over previous
import functools

import jax
import jax.numpy as jnp
from jax import lax
from jax.experimental import pallas as pl
from jax.experimental.pallas import tpu as pltpu
from jax.experimental.pallas import tpu_sc as plsc

VOCAB = 1000000
DHATU_VOCAB = 10000
EMBED_DIM = 64
HALF = 32

NC = 2
NS = 16
NW = NC * NS

N_TOKENS = 16384 * 50
PER_W = N_TOKENS // NW
CHUNK = 512
SUB = 128
N_CHUNKS = PER_W // CHUNK


def _sc_gather(ids, word_table, dhatu_table):
    mesh = plsc.VectorSubcoreMesh(core_axis_name="c", subcore_axis_name="s")

    @functools.partial(
        pl.kernel,
        out_type=(
            jax.ShapeDtypeStruct((N_TOKENS, HALF), jnp.float32),
            jax.ShapeDtypeStruct((N_TOKENS, HALF), jnp.float32),
        ),
        mesh=mesh,
        scratch_types=[
            pltpu.VMEM((PER_W,), jnp.int32),
            pltpu.VMEM((PER_W,), jnp.int32),
            pltpu.VMEM((CHUNK, HALF), jnp.float32),
            pltpu.VMEM((CHUNK, HALF), jnp.float32),
            pltpu.SemaphoreType.DMA,
            pltpu.SemaphoreType.DMA,
        ],
        compiler_params=pltpu.CompilerParams(use_tc_tiling_on_sc=False),
    )
    def gather_kernel(ids_hbm, wt_hbm, dt_hbm, wout_hbm, dout_hbm,
                      idx_v, didx_v, wrows, drows, wsem, dsem):
        wid = lax.axis_index("s") * NC + lax.axis_index("c")
        base = wid * PER_W
        pltpu.sync_copy(ids_hbm.at[pl.ds(base, PER_W)], idx_v)

        def mod_body(i, carry):
            v = idx_v[pl.ds(i * 16, 16)]
            didx_v[pl.ds(i * 16, 16)] = lax.rem(v, DHATU_VOCAB)
            return carry

        lax.fori_loop(0, PER_W // 16, mod_body, 0)

        def chunk_body(ci, carry):
            off = ci * CHUNK
            handles = []
            for j in range(CHUNK // SUB):
                hw = pltpu.async_copy(
                    wt_hbm.at[idx_v.at[pl.ds(off + j * SUB, SUB)]],
                    wrows.at[pl.ds(j * SUB, SUB)], wsem)
                hd = pltpu.async_copy(
                    dt_hbm.at[didx_v.at[pl.ds(off + j * SUB, SUB)]],
                    drows.at[pl.ds(j * SUB, SUB)], dsem)
                handles.append((hw, hd))
            for hw, hd in handles:
                hw.wait()
                hd.wait()
            pltpu.sync_copy(wrows, wout_hbm.at[pl.ds(base + off, CHUNK)])
            pltpu.sync_copy(drows, dout_hbm.at[pl.ds(base + off, CHUNK)])
            return carry

        lax.fori_loop(0, N_CHUNKS, chunk_body, 0)

    return gather_kernel(ids, word_table, dhatu_table)


_SQRT_HALF = 0.7071067811865476


def _dense_body(w_ref, d_ref, w1t_ref, w2t_ref, b_ref, g_ref, bt_ref, o_ref):
    y = (jnp.dot(w_ref[...], w1t_ref[...], preferred_element_type=jnp.float32)
         + jnp.dot(d_ref[...], w2t_ref[...], preferred_element_type=jnp.float32)
         + b_ref[...])
    mu = jnp.mean(y, axis=-1, keepdims=True)
    var = jnp.mean((y - mu) ** 2, axis=-1, keepdims=True)
    yn = (y - mu) / jnp.sqrt(var + 1e-5) * g_ref[...] + bt_ref[...]
    o_ref[...] = yn * 0.5 * (1.0 + lax.erf(yn * _SQRT_HALF))


def _tc_dense(w_emb, d_emb, W, b, gamma, beta):
    BK = 2048
    grid = (N_TOKENS // BK,)
    w1t = W[:, :HALF].T
    w2t = W[:, HALF:].T
    return pl.pallas_call(
        _dense_body,
        grid=grid,
        in_specs=[
            pl.BlockSpec((BK, HALF), lambda i: (i, 0)),
            pl.BlockSpec((BK, HALF), lambda i: (i, 0)),
            pl.BlockSpec((HALF, EMBED_DIM), lambda i: (0, 0)),
            pl.BlockSpec((HALF, EMBED_DIM), lambda i: (0, 0)),
            pl.BlockSpec((1, EMBED_DIM), lambda i: (0, 0)),
            pl.BlockSpec((1, EMBED_DIM), lambda i: (0, 0)),
            pl.BlockSpec((1, EMBED_DIM), lambda i: (0, 0)),
        ],
        out_specs=pl.BlockSpec((BK, EMBED_DIM), lambda i: (i, 0)),
        out_shape=jax.ShapeDtypeStruct((N_TOKENS, EMBED_DIM), jnp.float32),
    )(w_emb, d_emb, w1t, w2t, b.reshape(1, EMBED_DIM),
      gamma.reshape(1, EMBED_DIM), beta.reshape(1, EMBED_DIM))


def kernel(word_ids, word_table, dhatu_table, W, b, gamma, beta):
    B, L = word_ids.shape
    ids = word_ids.reshape(-1)
    w_emb, d_emb = _sc_gather(ids, word_table, dhatu_table)
    y = _tc_dense(w_emb, d_emb, W, b, gamma, beta)
    return y.reshape(B, L, EMBED_DIM)


# speedup vs baseline: 10.3857x; 1.0252x over previous
import functools

import jax
import jax.numpy as jnp
from jax import lax
from jax.experimental import pallas as pl
from jax.experimental.pallas import tpu as pltpu
from jax.experimental.pallas import tpu_sc as plsc

VOCAB = 1000000
DHATU_VOCAB = 10000
EMBED_DIM = 64
HALF = 32

NC = 2
NS = 16
NW = NC * NS

B, L = 16384, 50
N_TOKENS = B * L
B_PER_W = B // NW

TCH = 1600
N_TCH = VOCAB // TCH
K1_ITERS = (N_TCH + NW - 1) // NW
DCH = 400
N_DCH = DHATU_VOCAB // DCH

SUB = 128


def _sc_mesh():
    return plsc.VectorSubcoreMesh(core_axis_name="c", subcore_axis_name="s")


def _pack_body(x_ref, o_ref):
    t = x_ref[...].T
    t4 = t.reshape(t.shape[0] // 4, 4, HALF)
    o_ref[...] = jnp.concatenate([t4[:, k, :] for k in range(4)], axis=-1)


def _tc_pack(table_t, n_rows, blkw):
    grid = ((n_rows + blkw - 1) // blkw,)
    return pl.pallas_call(
        _pack_body,
        grid=grid,
        in_specs=[pl.BlockSpec((HALF, blkw), lambda i: (0, i))],
        out_specs=pl.BlockSpec((blkw // 4, 128), lambda i: (i, 0)),
        out_shape=jax.ShapeDtypeStruct((n_rows // 4, 128), jnp.float32),
    )(table_t)


def _k1_table_pack(wt_t, dt_t):
    w_packed = _tc_pack(wt_t, VOCAB, 8192)
    d_packed = _tc_pack(dt_t, DHATU_VOCAB, 10000)
    return (w_packed.reshape(VOCAB, HALF), d_packed.reshape(DHATU_VOCAB, HALF))


def _k2_gather(ids_lb, w_rm, d_rm):
    @functools.partial(
        pl.kernel,
        out_type=jax.ShapeDtypeStruct((L, EMBED_DIM, B), jnp.float32),
        mesh=_sc_mesh(),
        scratch_types=[
            pltpu.VMEM((B_PER_W,), jnp.int32),
            pltpu.VMEM((B_PER_W,), jnp.int32),
            pltpu.VMEM((B_PER_W, HALF), jnp.float32),
            pltpu.VMEM((B_PER_W, HALF), jnp.float32),
            pltpu.VMEM((HALF, B_PER_W), jnp.float32),
            pltpu.VMEM((HALF, B_PER_W), jnp.float32),
            pltpu.SemaphoreType.DMA,
            pltpu.SemaphoreType.DMA,
        ],
        compiler_params=pltpu.CompilerParams(use_tc_tiling_on_sc=False, needs_layout_passes=False),
    )
    def k2(ids_hbm, wt_hbm, dt_hbm, xout_hbm,
           idx_v, didx_v, wrows, drows, wtr, dtr, wsem, dsem):
        wid = lax.axis_index("s") * NC + lax.axis_index("c")
        b0 = wid * B_PER_W
        f16 = lax.iota(jnp.int32, 16)

        def l_body(l, carry):
            pltpu.sync_copy(ids_hbm.at[pl.ds(l * B + b0, B_PER_W)], idx_v)

            def mod_body(i, c2):
                v = idx_v[pl.ds(i * 16, 16)]
                didx_v[pl.ds(i * 16, 16)] = lax.rem(v, DHATU_VOCAB)
                return c2

            lax.fori_loop(0, B_PER_W // 16, mod_body, 0)

            handles = []
            for j in range(B_PER_W // SUB):
                hw = pltpu.async_copy(
                    wt_hbm.at[idx_v.at[pl.ds(j * SUB, SUB)]],
                    wrows.at[pl.ds(j * SUB, SUB)], wsem)
                hd = pltpu.async_copy(
                    dt_hbm.at[didx_v.at[pl.ds(j * SUB, SUB)]],
                    drows.at[pl.ds(j * SUB, SUB)], dsem)
                handles.append((hw, hd))
            for hw, hd in handles:
                hw.wait()
                hd.wait()

            def g_body(g, c2):
                rid = g * 16 + f16
                for f in range(HALF):
                    fv = jnp.full((16,), f, jnp.int32)
                    wtr[f, pl.ds(g * 16, 16)] = plsc.load_gather(
                        wrows, [rid, fv])
                    dtr[f, pl.ds(g * 16, 16)] = plsc.load_gather(
                        drows, [rid, fv])
                return c2

            lax.fori_loop(0, B_PER_W // 16, g_body, 0)

            pltpu.sync_copy(wtr, xout_hbm.at[l, pl.ds(0, HALF),
                                             pl.ds(b0, B_PER_W)])
            pltpu.sync_copy(dtr, xout_hbm.at[l, pl.ds(HALF, HALF),
                                             pl.ds(b0, B_PER_W)])
            return carry

        lax.fori_loop(0, L, l_body, 0)

    return k2(ids_lb, w_rm, d_rm)


_EPS = 1e-5


def _dense_body(x_ref, wc_ref, bc_ref, g_ref, bt_ref, o_ref):
    x = x_ref[0]
    yc = jnp.dot(wc_ref[...], x,
                 preferred_element_type=jnp.float32) + bc_ref[...]
    var = jnp.mean(yc * yc, axis=0, keepdims=True)
    r = 1.0 / jnp.sqrt(var + _EPS)
    yn = yc * r * g_ref[...] + bt_ref[...]
    o_ref[0] = yn * 0.5 * (1.0 + lax.erf(yn * 0.7071067811865476))


def _k3_dense(x_tr, Wc, bc, gamma, beta):
    BKB = 2048
    grid = (L, B // BKB)
    return pl.pallas_call(
        _dense_body,
        grid=grid,
        in_specs=[
            pl.BlockSpec((1, EMBED_DIM, BKB), lambda l, i: (l, 0, i)),
            pl.BlockSpec((EMBED_DIM, EMBED_DIM), lambda l, i: (0, 0)),
            pl.BlockSpec((EMBED_DIM, 1), lambda l, i: (0, 0)),
            pl.BlockSpec((EMBED_DIM, 1), lambda l, i: (0, 0)),
            pl.BlockSpec((EMBED_DIM, 1), lambda l, i: (0, 0)),
        ],
        out_specs=pl.BlockSpec((1, EMBED_DIM, BKB), lambda l, i: (l, 0, i)),
        out_shape=jax.ShapeDtypeStruct((L, EMBED_DIM, B), jnp.float32),
    )(x_tr, Wc, bc, gamma, beta)


def kernel(word_ids, word_table, dhatu_table, W, b, gamma, beta):
    wt_t = word_table.T
    dt_t = dhatu_table.T
    ids_lb = word_ids.T.reshape(-1)

    w_rm, d_rm = _k1_table_pack(wt_t, dt_t)
    x_tr = _k2_gather(ids_lb, w_rm, d_rm)

    C = jnp.eye(EMBED_DIM, dtype=jnp.float32) - 1.0 / EMBED_DIM
    Wc = C @ W
    bc = (C @ b).reshape(EMBED_DIM, 1)
    ytr = _k3_dense(x_tr, Wc, bc,
                    gamma.reshape(EMBED_DIM, 1), beta.reshape(EMBED_DIM, 1))
    return ytr.transpose(2, 0, 1)


# speedup vs baseline: 15.6342x; 1.5054x over previous
import functools

import jax
import jax.numpy as jnp
from jax import lax
from jax.experimental import pallas as pl
from jax.experimental.pallas import tpu as pltpu
from jax.experimental.pallas import tpu_sc as plsc

VOCAB = 1000000
DHATU_VOCAB = 10000
EMBED_DIM = 64
HALF = 32

NC = 2
NS = 16
NW = NC * NS

B, L = 16384, 50
N_TOKENS = B * L
B_PER_W = B // NW

TCH = 1600
N_TCH = VOCAB // TCH
K1_ITERS = (N_TCH + NW - 1) // NW
DCH = 400
N_DCH = DHATU_VOCAB // DCH

SUB = 128


def _sc_mesh():
    return plsc.VectorSubcoreMesh(core_axis_name="c", subcore_axis_name="s")


def _pack_body(x_ref, o_ref):
    t = x_ref[...].T
    t4 = t.reshape(t.shape[0] // 4, 4, HALF)
    o_ref[...] = jnp.concatenate([t4[:, k, :] for k in range(4)], axis=-1)


def _tc_pack(table_t, n_rows, blkw):
    grid = ((n_rows + blkw - 1) // blkw,)
    return pl.pallas_call(
        _pack_body,
        grid=grid,
        in_specs=[pl.BlockSpec((HALF, blkw), lambda i: (0, i))],
        out_specs=pl.BlockSpec((blkw // 4, 128), lambda i: (i, 0)),
        out_shape=jax.ShapeDtypeStruct((n_rows // 4, 128), jnp.float32),
    )(table_t)


def _k1_table_pack(wt_t, dt_t):
    w_packed = _tc_pack(wt_t, VOCAB, 8192)
    d_packed = _tc_pack(dt_t, DHATU_VOCAB, 10000)
    return (w_packed.reshape(VOCAB, HALF), d_packed.reshape(DHATU_VOCAB, HALF))


TRS = B_PER_W + 17


def _k2_gather(ids_lb, w_rm, d_rm):
    @functools.partial(
        pl.kernel,
        out_type=jax.ShapeDtypeStruct((L * EMBED_DIM, B), jnp.float32),
        mesh=_sc_mesh(),
        scratch_types=[
            pltpu.VMEM((B_PER_W,), jnp.int32),
            pltpu.VMEM((B_PER_W,), jnp.int32),
            pltpu.VMEM((B_PER_W, HALF), jnp.float32),
            pltpu.VMEM((B_PER_W, HALF), jnp.float32),
            pltpu.VMEM((HALF, TRS), jnp.float32),
            pltpu.VMEM((HALF, TRS), jnp.float32),
            pltpu.SemaphoreType.DMA,
            pltpu.SemaphoreType.DMA,
        ],
        compiler_params=pltpu.CompilerParams(use_tc_tiling_on_sc=False, needs_layout_passes=False),
    )
    def k2(ids_hbm, wt_hbm, dt_hbm, xout_hbm,
           idx_v, didx_v, wrows, drows, wtr, dtr, wsem, dsem):
        wid = lax.axis_index("s") * NC + lax.axis_index("c")
        b0 = wid * B_PER_W
        f16 = lax.iota(jnp.int32, 16)
        f16h = f16 + 16

        def l_body(l, carry):
            pltpu.sync_copy(ids_hbm.at[pl.ds(l * B + b0, B_PER_W)], idx_v)

            def mod_body(i, c2):
                v = idx_v[pl.ds(i * 16, 16)]
                didx_v[pl.ds(i * 16, 16)] = lax.rem(v, DHATU_VOCAB)
                return c2

            lax.fori_loop(0, B_PER_W // 16, mod_body, 0)

            handles = []
            for j in range(B_PER_W // SUB):
                hw = pltpu.async_copy(
                    wt_hbm.at[idx_v.at[pl.ds(j * SUB, SUB)]],
                    wrows.at[pl.ds(j * SUB, SUB)], wsem)
                hd = pltpu.async_copy(
                    dt_hbm.at[didx_v.at[pl.ds(j * SUB, SUB)]],
                    drows.at[pl.ds(j * SUB, SUB)], dsem)
                handles.append((hw, hd))
            for hw, hd in handles:
                hw.wait()
                hd.wait()

            def t_body(t8, c2):
                for dt in range(8):
                    t = t8 * 8 + dt
                    tv = jnp.full((16,), 0, jnp.int32) + t
                    plsc.store_scatter(wtr, [f16, tv], wrows[t, pl.ds(0, 16)])
                    plsc.store_scatter(wtr, [f16h, tv], wrows[t, pl.ds(16, 16)])
                    plsc.store_scatter(dtr, [f16, tv], drows[t, pl.ds(0, 16)])
                    plsc.store_scatter(dtr, [f16h, tv], drows[t, pl.ds(16, 16)])
                return c2

            lax.fori_loop(0, B_PER_W // 8, t_body, 0)

            pltpu.sync_copy(wtr.at[:, pl.ds(0, B_PER_W)],
                            xout_hbm.at[pl.ds(l * EMBED_DIM, HALF),
                                        pl.ds(b0, B_PER_W)])
            pltpu.sync_copy(dtr.at[:, pl.ds(0, B_PER_W)],
                            xout_hbm.at[pl.ds(l * EMBED_DIM + HALF, HALF),
                                        pl.ds(b0, B_PER_W)])
            return carry

        lax.fori_loop(0, L, l_body, 0)

    return k2(ids_lb, w_rm, d_rm)


_EPS = 1e-5


def _dense_body(x_ref, wc_ref, bc_ref, g_ref, bt_ref, o_ref):
    x = x_ref[...]
    yc = jnp.dot(wc_ref[...], x, preferred_element_type=jnp.float32,
                 precision=lax.Precision.HIGHEST) + bc_ref[...]
    var = jnp.mean(yc * yc, axis=0, keepdims=True)
    r = 1.0 / jnp.sqrt(var + _EPS)
    yn = yc * r * g_ref[...] + bt_ref[...]
    o_ref[...] = yn * 0.5 * (1.0 + lax.erf(yn * 0.7071067811865476))


def _k3_dense(x_tr, Wc, bc, gamma, beta):
    BKB = 2048
    grid = (L, B // BKB)
    return pl.pallas_call(
        _dense_body,
        grid=grid,
        in_specs=[
            pl.BlockSpec((EMBED_DIM, BKB), lambda l, i: (l, i)),
            pl.BlockSpec((EMBED_DIM, EMBED_DIM), lambda l, i: (0, 0)),
            pl.BlockSpec((EMBED_DIM, 1), lambda l, i: (0, 0)),
            pl.BlockSpec((EMBED_DIM, 1), lambda l, i: (0, 0)),
            pl.BlockSpec((EMBED_DIM, 1), lambda l, i: (0, 0)),
        ],
        out_specs=pl.BlockSpec((EMBED_DIM, BKB), lambda l, i: (l, i)),
        out_shape=jax.ShapeDtypeStruct((L * EMBED_DIM, B), jnp.float32),
    )(x_tr, Wc, bc, gamma, beta)


def kernel(word_ids, word_table, dhatu_table, W, b, gamma, beta):
    wt_t = word_table.T
    dt_t = dhatu_table.T
    ids_lb = word_ids.T.reshape(-1)

    w_rm, d_rm = _k1_table_pack(wt_t, dt_t)
    x_tr = _k2_gather(ids_lb, w_rm, d_rm)

    C = jnp.eye(EMBED_DIM, dtype=jnp.float32) - 1.0 / EMBED_DIM
    Wc = C @ W
    bc = (C @ b).reshape(EMBED_DIM, 1)
    ytr = _k3_dense(x_tr, Wc, bc,
                    gamma.reshape(EMBED_DIM, 1), beta.reshape(EMBED_DIM, 1))
    return ytr.reshape(L, EMBED_DIM, B).transpose(2, 0, 1)
